# Initial kernel scaffold; baseline (speedup 1.0000x reference)
import jax
import jax.numpy as jnp
from jax.experimental import pallas as pl


def kernel(x, edge_index, W1, b1, g1, bt1, W2, b2, g2, bt2):
    raise NotImplementedError("write your pallas kernel here")


# speedup vs baseline: 12.6944x; 12.6944x over previous
import functools

import jax
import jax.numpy as jnp
from jax import lax
from jax.experimental import pallas as pl
from jax.experimental.pallas import tpu as pltpu
from jax.experimental.pallas import tpu_sc as plsc

N = 10000
E = 320000
H = 128
EPS = 1e-5

NC = 2
NS = 16
EPC = E // NC
EPT = EPC // NS
CH = 80
NCHUNK = EPT // CH
NPAD = 10240
RPT = NPAD // NS
ZR = 128
CPT = NPAD // NS

ROWBLK = 1000
GRID = N // ROWBLK


def _deg_partials(dst):
    mesh = plsc.VectorSubcoreMesh(core_axis_name="c", subcore_axis_name="s")

    @functools.partial(
        pl.kernel,
        mesh=mesh,
        out_type=jax.ShapeDtypeStruct((NC * NPAD,), jnp.float32),
        scratch_types=[
            pltpu.VMEM((CH,), jnp.int32),
            pltpu.VMEM((CH,), jnp.float32),
            pltpu.VMEM((CPT,), jnp.float32),
            pltpu.VMEM_SHARED((NPAD,), jnp.float32),
        ],
    )
    def deg_kernel(dst_hbm, out_hbm, idx_d, ones_v, zbuf, cnt):
        core = lax.axis_index("c")
        sid = lax.axis_index("s")
        for k in range(CPT // 16):
            zbuf[pl.ds(16 * k, 16)] = jnp.zeros((16,), jnp.float32)
        for k in range(CH // 16):
            ones_v[pl.ds(16 * k, 16)] = jnp.ones((16,), jnp.float32)
        base_c = pl.multiple_of(sid * CPT, 8)
        pltpu.sync_copy(zbuf, cnt.at[pl.ds(base_c, CPT)])
        plsc.subcore_barrier()
        ebase = core * EPC + sid * EPT

        def body(j, carry):
            b = pl.multiple_of(ebase + j * CH, 8)
            pltpu.sync_copy(dst_hbm.at[pl.ds(b, CH)], idx_d)
            pltpu.sync_copy(ones_v, cnt.at[idx_d], add=True)
            return carry

        lax.fori_loop(0, NCHUNK, body, 0)
        plsc.subcore_barrier()
        pltpu.sync_copy(cnt.at[pl.ds(base_c, CPT)], zbuf)
        obase = pl.multiple_of(core * NPAD + sid * CPT, 8)
        pltpu.sync_copy(zbuf, out_hbm.at[pl.ds(obase, CPT)])

    return deg_kernel(dst).reshape(NC, NPAD)


def _scatter_partials(u, src, dst):
    mesh = plsc.VectorSubcoreMesh(core_axis_name="c", subcore_axis_name="s")

    @functools.partial(
        pl.kernel,
        mesh=mesh,
        out_type=jax.ShapeDtypeStruct((NC, NPAD, H), jnp.float32),
        scratch_types=[
            pltpu.VMEM((CH,), jnp.int32),
            pltpu.VMEM((CH,), jnp.int32),
            pltpu.VMEM((CH, H), jnp.float32),
            pltpu.VMEM((ZR, H), jnp.float32),
            pltpu.VMEM_SHARED((NPAD, H), jnp.float32),
            pltpu.SemaphoreType.DMA,
        ],
    )
    def scatter_kernel(u_hbm, src_hbm, dst_hbm, out_hbm, idx_s, idx_d, rows,
                       zbuf, acc, sem):
        core = lax.axis_index("c")
        sid = lax.axis_index("s")

        def zrow(i, carry):
            for k in range(H // 16):
                zbuf[i, pl.ds(16 * k, 16)] = jnp.zeros((16,), jnp.float32)
            return carry

        lax.fori_loop(0, ZR, zrow, 0)
        row0 = pl.multiple_of(sid * RPT, 8)
        for k in range(RPT // ZR):
            pltpu.sync_copy(zbuf, acc.at[pl.ds(row0 + k * ZR, ZR)])
        plsc.subcore_barrier()

        ebase = core * EPC + sid * EPT

        def body(j, carry):
            b = pl.multiple_of(ebase + j * CH, 8)
            pltpu.sync_copy(src_hbm.at[pl.ds(b, CH)], idx_s)
            pltpu.sync_copy(dst_hbm.at[pl.ds(b, CH)], idx_d)
            pltpu.async_copy(u_hbm.at[idx_s], rows, sem).wait()
            pltpu.sync_copy(rows, acc.at[idx_d], add=True)
            return carry

        lax.fori_loop(0, NCHUNK, body, 0)
        plsc.subcore_barrier()
        for k in range(RPT // ZR):
            pltpu.sync_copy(acc.at[pl.ds(row0 + k * ZR, ZR)], zbuf)
            pltpu.sync_copy(zbuf, out_hbm.at[core, pl.ds(row0 + k * ZR, ZR)])

    return scatter_kernel(u, src, dst)[:, :N, :]


def _dscale(c_ref):
    deg = c_ref[:, 0] + c_ref[:, 1] + 1.0
    return lax.rsqrt(deg)


def _mm_body(x_ref, w_ref, o_ref):
    o_ref[...] = jnp.dot(x_ref[...], w_ref[...],
                         preferred_element_type=jnp.float32,
                         precision=lax.Precision.HIGHEST)


def _matmul(x, W):
    return pl.pallas_call(
        _mm_body,
        grid=(GRID,),
        in_specs=[
            pl.BlockSpec((ROWBLK, H), lambda i: (i, 0)),
            pl.BlockSpec((H, H), lambda i: (0, 0)),
        ],
        out_specs=pl.BlockSpec((ROWBLK, H), lambda i: (i, 0)),
        out_shape=jax.ShapeDtypeStruct((N, H), jnp.float32),
    )(x, W)


def _scale_body(h_ref, c_ref, o_ref):
    o_ref[...] = h_ref[...] * _dscale(c_ref)[:, None]


def _scale(h, counts):
    return pl.pallas_call(
        _scale_body,
        grid=(GRID,),
        in_specs=[
            pl.BlockSpec((ROWBLK, H), lambda i: (i, 0)),
            pl.BlockSpec((ROWBLK, NC), lambda i: (i, 0)),
        ],
        out_specs=pl.BlockSpec((ROWBLK, H), lambda i: (i, 0)),
        out_shape=jax.ShapeDtypeStruct((N, H), jnp.float32),
    )(h, counts)


def _combine_body(s_ref, u_ref, c_ref, b_ref, a_ref, st_ref):
    i = pl.program_id(0)
    d = _dscale(c_ref)
    s = s_ref[0] + s_ref[1] + u_ref[...]
    a = s * d[:, None] + b_ref[...]
    a_ref[...] = a
    cs = jnp.sum(a, axis=0, keepdims=True)
    cq = jnp.sum(a * a, axis=0, keepdims=True)
    st = jnp.concatenate([cs, cq], axis=0)

    @pl.when(i == 0)
    def _():
        st_ref[...] = st

    @pl.when(i > 0)
    def _():
        st_ref[...] = st_ref[...] + st


def _combine(S, u, counts, b):
    return pl.pallas_call(
        _combine_body,
        grid=(GRID,),
        in_specs=[
            pl.BlockSpec((NC, ROWBLK, H), lambda i: (0, i, 0)),
            pl.BlockSpec((ROWBLK, H), lambda i: (i, 0)),
            pl.BlockSpec((ROWBLK, NC), lambda i: (i, 0)),
            pl.BlockSpec((1, H), lambda i: (0, 0)),
        ],
        out_specs=[
            pl.BlockSpec((ROWBLK, H), lambda i: (i, 0)),
            pl.BlockSpec((2, H), lambda i: (0, 0)),
        ],
        out_shape=[
            jax.ShapeDtypeStruct((N, H), jnp.float32),
            jax.ShapeDtypeStruct((2, H), jnp.float32),
        ],
    )(S, u, counts, b)


def _bn_relu(a_ref, st_ref, g_ref, bt_ref):
    mean = st_ref[0, :] * (1.0 / N)
    var = st_ref[1, :] * (1.0 / N) - mean * mean
    inv = lax.rsqrt(var + EPS)
    y = (a_ref[...] - mean[None, :]) * (inv[None, :] * g_ref[...]) + bt_ref[...]
    return jnp.maximum(y, 0.0)


def _normmm_body(a_ref, st_ref, g_ref, bt_ref, w_ref, c_ref, o_ref):
    y = _bn_relu(a_ref, st_ref, g_ref, bt_ref)
    hh = jnp.dot(y, w_ref[...], preferred_element_type=jnp.float32,
                 precision=lax.Precision.HIGHEST)
    o_ref[...] = hh * _dscale(c_ref)[:, None]


def _norm_matmul_scale(a, st, g, bt, W, counts):
    return pl.pallas_call(
        _normmm_body,
        grid=(GRID,),
        in_specs=[
            pl.BlockSpec((ROWBLK, H), lambda i: (i, 0)),
            pl.BlockSpec((2, H), lambda i: (0, 0)),
            pl.BlockSpec((1, H), lambda i: (0, 0)),
            pl.BlockSpec((1, H), lambda i: (0, 0)),
            pl.BlockSpec((H, H), lambda i: (0, 0)),
            pl.BlockSpec((ROWBLK, NC), lambda i: (i, 0)),
        ],
        out_specs=pl.BlockSpec((ROWBLK, H), lambda i: (i, 0)),
        out_shape=jax.ShapeDtypeStruct((N, H), jnp.float32),
    )(a, st, g, bt, W, counts)


def _final_body(a_ref, st_ref, g_ref, bt_ref, o_ref):
    o_ref[...] = _bn_relu(a_ref, st_ref, g_ref, bt_ref)


def _final_norm(a, st, g, bt):
    return pl.pallas_call(
        _final_body,
        grid=(GRID,),
        in_specs=[
            pl.BlockSpec((ROWBLK, H), lambda i: (i, 0)),
            pl.BlockSpec((2, H), lambda i: (0, 0)),
            pl.BlockSpec((1, H), lambda i: (0, 0)),
            pl.BlockSpec((1, H), lambda i: (0, 0)),
        ],
        out_specs=pl.BlockSpec((ROWBLK, H), lambda i: (i, 0)),
        out_shape=jax.ShapeDtypeStruct((N, H), jnp.float32),
    )(a, st, g, bt)


def kernel(x, edge_index, W1, b1, g1, bt1, W2, b2, g2, bt2):
    src = edge_index[0]
    dst = edge_index[1]
    counts = _deg_partials(dst)[:, :N].T

    h1 = _matmul(x, W1)
    u1 = _scale(h1, counts)
    S1 = _scatter_partials(u1, src, dst)
    a1, st1 = _combine(S1, u1, counts, b1.reshape(1, H))

    u2 = _norm_matmul_scale(a1, st1, g1.reshape(1, H), bt1.reshape(1, H), W2,
                            counts)
    S2 = _scatter_partials(u2, src, dst)
    a2, st2 = _combine(S2, u2, counts, b2.reshape(1, H))

    return _final_norm(a2, st2, g2.reshape(1, H), bt2.reshape(1, H))


# speedup vs baseline: 21.3495x; 1.6818x over previous
import functools

import jax
import jax.numpy as jnp
from jax import lax
from jax.experimental import pallas as pl
from jax.experimental.pallas import tpu as pltpu
from jax.experimental.pallas import tpu_sc as plsc

N = 10000
E = 320000
H = 128
EPS = 1e-5

NC = 2
NS = 16
EPC = E // NC
EPT = EPC // NS
CH = 80
NCHUNK = EPT // CH
NPAD = 10240
RPT = NPAD // NS
ZR = 128
CPT = NPAD // NS

ROWBLK = 1000
GRID = N // ROWBLK


def _deg_partials(dst):
    mesh = plsc.VectorSubcoreMesh(core_axis_name="c", subcore_axis_name="s")

    @functools.partial(
        pl.kernel,
        mesh=mesh,
        out_type=jax.ShapeDtypeStruct((NC * NPAD,), jnp.float32),
        scratch_types=[
            pltpu.VMEM((CH,), jnp.int32),
            pltpu.VMEM((CH,), jnp.int32),
            pltpu.VMEM((CH,), jnp.float32),
            pltpu.VMEM((CPT,), jnp.float32),
            pltpu.VMEM_SHARED((NPAD,), jnp.float32),
            pltpu.SemaphoreType.DMA,
        ],
    )
    def deg_kernel(dst_hbm, out_hbm, idx0, idx1, ones_v, zbuf, cnt, isem):
        core = lax.axis_index("c")
        sid = lax.axis_index("s")
        for k in range(CPT // 16):
            zbuf[pl.ds(16 * k, 16)] = jnp.zeros((16,), jnp.float32)
        for k in range(CH // 16):
            ones_v[pl.ds(16 * k, 16)] = jnp.ones((16,), jnp.float32)
        base_c = pl.multiple_of(sid * CPT, 8)
        pltpu.sync_copy(zbuf, cnt.at[pl.ds(base_c, CPT)])
        ebase = pl.multiple_of(core * EPC + sid * EPT, 8)
        plsc.subcore_barrier()

        def idx_dma(j, ibuf):
            b = pl.multiple_of(ebase + j * CH, 8)
            return pltpu.make_async_copy(dst_hbm.at[pl.ds(b, CH)], ibuf, isem)

        def step(j, icur, ioth):
            idx_dma(j, icur).wait()

            @pl.when(j < NCHUNK - 1)
            def _():
                idx_dma(j + 1, ioth).start()

            pltpu.sync_copy(ones_v, cnt.at[icur], add=True)

        idx_dma(0, idx0).start()

        def body(t, carry):
            step(2 * t, idx0, idx1)
            step(2 * t + 1, idx1, idx0)
            return carry

        lax.fori_loop(0, (NCHUNK - 1) // 2, body, 0)
        step(NCHUNK - 1, idx0, idx1)
        plsc.subcore_barrier()
        pltpu.sync_copy(cnt.at[pl.ds(base_c, CPT)], zbuf)
        obase = pl.multiple_of(core * NPAD + sid * CPT, 8)
        pltpu.sync_copy(zbuf, out_hbm.at[pl.ds(obase, CPT)])

    return deg_kernel(dst).reshape(NC, NPAD)


def _scatter_partials(u, src, dst):
    mesh = plsc.VectorSubcoreMesh(core_axis_name="c", subcore_axis_name="s")

    @functools.partial(
        pl.kernel,
        mesh=mesh,
        out_type=jax.ShapeDtypeStruct((NC, NPAD, H), jnp.float32),
        scratch_types=[
            pltpu.VMEM((EPT,), jnp.int32),
            pltpu.VMEM((CH,), jnp.int32),
            pltpu.VMEM((CH,), jnp.int32),
            pltpu.VMEM((CH, H), jnp.float32),
            pltpu.VMEM((CH, H), jnp.float32),
            pltpu.VMEM((ZR, H), jnp.float32),
            pltpu.VMEM_SHARED((NPAD, H), jnp.float32),
            pltpu.SemaphoreType.DMA,
            pltpu.SemaphoreType.DMA,
        ],
    )
    def scatter_kernel(u_hbm, src_hbm, dst_hbm, out_hbm, sidx, idx_d0, idx_d1,
                       rows0, rows1, zbuf, acc, gsem, isem):
        core = lax.axis_index("c")
        sid = lax.axis_index("s")

        def zrow(i, carry):
            for k in range(H // 16):
                zbuf[i, pl.ds(16 * k, 16)] = jnp.zeros((16,), jnp.float32)
            return carry

        lax.fori_loop(0, ZR, zrow, 0)
        row0 = pl.multiple_of(sid * RPT, 8)
        for k in range(RPT // ZR):
            pltpu.sync_copy(zbuf, acc.at[pl.ds(row0 + k * ZR, ZR)])
        ebase = pl.multiple_of(core * EPC + sid * EPT, 8)
        pltpu.sync_copy(src_hbm.at[pl.ds(ebase, EPT)], sidx)
        plsc.subcore_barrier()

        def gather_of(j, rbuf):
            b = pl.multiple_of(j * CH, 8)
            return pltpu.make_async_copy(u_hbm.at[sidx.at[pl.ds(b, CH)]],
                                         rbuf, gsem)

        def idx_dma(j, ibuf):
            b = pl.multiple_of(ebase + j * CH, 8)
            return pltpu.make_async_copy(dst_hbm.at[pl.ds(b, CH)], ibuf, isem)

        def step(j, rcur, roth, icur, ioth):
            gather_of(j, rcur).wait()

            @pl.when(j < NCHUNK - 1)
            def _():
                gather_of(j + 1, roth).start()
                idx_dma(j + 1, ioth).start()

            idx_dma(j, icur).wait()
            pltpu.sync_copy(rcur, acc.at[icur], add=True)

        gather_of(0, rows0).start()
        idx_dma(0, idx_d0).start()

        def pair(t, carry):
            step(2 * t, rows0, rows1, idx_d0, idx_d1)
            step(2 * t + 1, rows1, rows0, idx_d1, idx_d0)
            return carry

        lax.fori_loop(0, (NCHUNK - 1) // 2, pair, 0)
        step(NCHUNK - 1, rows0, rows1, idx_d0, idx_d1)
        plsc.subcore_barrier()
        for k in range(RPT // ZR):
            pltpu.sync_copy(acc.at[pl.ds(row0 + k * ZR, ZR)], zbuf)
            pltpu.sync_copy(zbuf, out_hbm.at[core, pl.ds(row0 + k * ZR, ZR)])

    return scatter_kernel(u, src, dst)[:, :N, :]


def _dscale(c_ref):
    deg = c_ref[:, 0] + c_ref[:, 1] + 1.0
    return lax.rsqrt(deg)


def _mm_body(x_ref, w_ref, o_ref):
    o_ref[...] = jnp.dot(x_ref[...], w_ref[...],
                         preferred_element_type=jnp.float32,
                         precision=lax.Precision.HIGHEST)


def _matmul(x, W):
    return pl.pallas_call(
        _mm_body,
        grid=(GRID,),
        in_specs=[
            pl.BlockSpec((ROWBLK, H), lambda i: (i, 0)),
            pl.BlockSpec((H, H), lambda i: (0, 0)),
        ],
        out_specs=pl.BlockSpec((ROWBLK, H), lambda i: (i, 0)),
        out_shape=jax.ShapeDtypeStruct((N, H), jnp.float32),
    )(x, W)


def _scale_body(h_ref, c_ref, o_ref):
    o_ref[...] = h_ref[...] * _dscale(c_ref)[:, None]


def _scale(h, counts):
    return pl.pallas_call(
        _scale_body,
        grid=(GRID,),
        in_specs=[
            pl.BlockSpec((ROWBLK, H), lambda i: (i, 0)),
            pl.BlockSpec((ROWBLK, NC), lambda i: (i, 0)),
        ],
        out_specs=pl.BlockSpec((ROWBLK, H), lambda i: (i, 0)),
        out_shape=jax.ShapeDtypeStruct((N, H), jnp.float32),
    )(h, counts)


def _combine_body(s_ref, u_ref, c_ref, b_ref, a_ref, st_ref):
    i = pl.program_id(0)
    d = _dscale(c_ref)
    s = s_ref[0] + s_ref[1] + u_ref[...]
    a = s * d[:, None] + b_ref[...]
    a_ref[...] = a
    cs = jnp.sum(a, axis=0, keepdims=True)
    cq = jnp.sum(a * a, axis=0, keepdims=True)
    st = jnp.concatenate([cs, cq], axis=0)

    @pl.when(i == 0)
    def _():
        st_ref[...] = st

    @pl.when(i > 0)
    def _():
        st_ref[...] = st_ref[...] + st


def _combine(S, u, counts, b):
    return pl.pallas_call(
        _combine_body,
        grid=(GRID,),
        in_specs=[
            pl.BlockSpec((NC, ROWBLK, H), lambda i: (0, i, 0)),
            pl.BlockSpec((ROWBLK, H), lambda i: (i, 0)),
            pl.BlockSpec((ROWBLK, NC), lambda i: (i, 0)),
            pl.BlockSpec((1, H), lambda i: (0, 0)),
        ],
        out_specs=[
            pl.BlockSpec((ROWBLK, H), lambda i: (i, 0)),
            pl.BlockSpec((2, H), lambda i: (0, 0)),
        ],
        out_shape=[
            jax.ShapeDtypeStruct((N, H), jnp.float32),
            jax.ShapeDtypeStruct((2, H), jnp.float32),
        ],
    )(S, u, counts, b)


def _bn_relu(a_ref, st_ref, g_ref, bt_ref):
    mean = st_ref[0, :] * (1.0 / N)
    var = st_ref[1, :] * (1.0 / N) - mean * mean
    inv = lax.rsqrt(var + EPS)
    y = (a_ref[...] - mean[None, :]) * (inv[None, :] * g_ref[...]) + bt_ref[...]
    return jnp.maximum(y, 0.0)


def _normmm_body(a_ref, st_ref, g_ref, bt_ref, w_ref, c_ref, o_ref):
    y = _bn_relu(a_ref, st_ref, g_ref, bt_ref)
    hh = jnp.dot(y, w_ref[...], preferred_element_type=jnp.float32,
                 precision=lax.Precision.HIGHEST)
    o_ref[...] = hh * _dscale(c_ref)[:, None]


def _norm_matmul_scale(a, st, g, bt, W, counts):
    return pl.pallas_call(
        _normmm_body,
        grid=(GRID,),
        in_specs=[
            pl.BlockSpec((ROWBLK, H), lambda i: (i, 0)),
            pl.BlockSpec((2, H), lambda i: (0, 0)),
            pl.BlockSpec((1, H), lambda i: (0, 0)),
            pl.BlockSpec((1, H), lambda i: (0, 0)),
            pl.BlockSpec((H, H), lambda i: (0, 0)),
            pl.BlockSpec((ROWBLK, NC), lambda i: (i, 0)),
        ],
        out_specs=pl.BlockSpec((ROWBLK, H), lambda i: (i, 0)),
        out_shape=jax.ShapeDtypeStruct((N, H), jnp.float32),
    )(a, st, g, bt, W, counts)


def _final_body(a_ref, st_ref, g_ref, bt_ref, o_ref):
    o_ref[...] = _bn_relu(a_ref, st_ref, g_ref, bt_ref)


def _final_norm(a, st, g, bt):
    return pl.pallas_call(
        _final_body,
        grid=(GRID,),
        in_specs=[
            pl.BlockSpec((ROWBLK, H), lambda i: (i, 0)),
            pl.BlockSpec((2, H), lambda i: (0, 0)),
            pl.BlockSpec((1, H), lambda i: (0, 0)),
            pl.BlockSpec((1, H), lambda i: (0, 0)),
        ],
        out_specs=pl.BlockSpec((ROWBLK, H), lambda i: (i, 0)),
        out_shape=jax.ShapeDtypeStruct((N, H), jnp.float32),
    )(a, st, g, bt)


def kernel(x, edge_index, W1, b1, g1, bt1, W2, b2, g2, bt2):
    src = edge_index[0]
    dst = edge_index[1]
    counts = _deg_partials(dst)[:, :N].T

    h1 = _matmul(x, W1)
    u1 = _scale(h1, counts)
    S1 = _scatter_partials(u1, src, dst)
    a1, st1 = _combine(S1, u1, counts, b1.reshape(1, H))

    u2 = _norm_matmul_scale(a1, st1, g1.reshape(1, H), bt1.reshape(1, H), W2,
                            counts)
    S2 = _scatter_partials(u2, src, dst)
    a2, st2 = _combine(S2, u2, counts, b2.reshape(1, H))

    return _final_norm(a2, st2, g2.reshape(1, H), bt2.reshape(1, H))
